# baseline (speedup 1.0000x reference)
import functools

import jax
import jax.numpy as jnp
from jax.experimental import pallas as pl

N_BLK = 1000
E_ROWS = 10000
E_2D = 500


def _dense_body(x_ref, w_ref, asrc_ref, adst_ref, bprev_ref, h_ref, al_ref, ar_ref, *, apply_relu):
    xb = x_ref[...]
    if apply_relu:
        xb = jnp.maximum(xb + bprev_ref[...], 0.0)
    h = jnp.dot(xb, w_ref[...], preferred_element_type=jnp.float32, precision=jax.lax.Precision.HIGHEST)
    h_ref[...] = h
    al_ref[...] = jnp.dot(h, asrc_ref[...], preferred_element_type=jnp.float32, precision=jax.lax.Precision.HIGHEST)
    ar_ref[...] = jnp.dot(h, adst_ref[...], preferred_element_type=jnp.float32, precision=jax.lax.Precision.HIGHEST)


def _dense(xin, W, a_src, a_dst, b_prev, apply_relu):
    n, d_in = xin.shape
    d_out = W.shape[1]
    grid = n // N_BLK
    body = functools.partial(_dense_body, apply_relu=apply_relu)
    return pl.pallas_call(
        body,
        grid=(grid,),
        in_specs=[
            pl.BlockSpec((N_BLK, d_in), lambda i: (i, 0)),
            pl.BlockSpec((d_in, d_out), lambda i: (0, 0)),
            pl.BlockSpec((d_out, 1), lambda i: (0, 0)),
            pl.BlockSpec((d_out, 1), lambda i: (0, 0)),
            pl.BlockSpec((1, d_out), lambda i: (0, 0)),
        ],
        out_specs=[
            pl.BlockSpec((N_BLK, d_out), lambda i: (i, 0)),
            pl.BlockSpec((N_BLK, 1), lambda i: (i, 0)),
            pl.BlockSpec((N_BLK, 1), lambda i: (i, 0)),
        ],
        out_shape=[
            jax.ShapeDtypeStruct((n, d_out), jnp.float32),
            jax.ShapeDtypeStruct((n, 1), jnp.float32),
            jax.ShapeDtypeStruct((n, 1), jnp.float32),
        ],
    )(xin, W, a_src.reshape(d_out, 1), a_dst.reshape(d_out, 1), b_prev.reshape(1, d_out))


def _edge_logit_body(alg_ref, arg_ref, e_ref):
    t = alg_ref[...] + arg_ref[...]
    e_ref[...] = jnp.where(t >= 0.0, t, 0.2 * t)


def _edge_exp_body(e_ref, mg_ref, ex_ref):
    ex_ref[...] = jnp.exp(e_ref[...] - mg_ref[...])


def _edge_elementwise(body, a, b):
    rows = a.shape[0] // E_2D
    a2 = a.reshape(rows, E_2D)
    b2 = b.reshape(rows, E_2D)
    blk_rows = 200
    out = pl.pallas_call(
        body,
        grid=(rows // blk_rows,),
        in_specs=[
            pl.BlockSpec((blk_rows, E_2D), lambda i: (i, 0)),
            pl.BlockSpec((blk_rows, E_2D), lambda i: (i, 0)),
        ],
        out_specs=pl.BlockSpec((blk_rows, E_2D), lambda i: (i, 0)),
        out_shape=jax.ShapeDtypeStruct((rows, E_2D), jnp.float32),
    )(a2, b2)
    return out.reshape(-1)


def _msg_body(hg_ref, ex_ref, sg_ref, out_ref):
    alpha = ex_ref[...] / (sg_ref[...] + 1e-16)
    out_ref[...] = hg_ref[...] * alpha


def _messages(h_g, ex, s_g):
    e_total, d = h_g.shape
    grid = e_total // E_ROWS
    return pl.pallas_call(
        _msg_body,
        grid=(grid,),
        in_specs=[
            pl.BlockSpec((E_ROWS, d), lambda i: (i, 0)),
            pl.BlockSpec((E_ROWS, 1), lambda i: (i, 0)),
            pl.BlockSpec((E_ROWS, 1), lambda i: (i, 0)),
        ],
        out_specs=pl.BlockSpec((E_ROWS, d), lambda i: (i, 0)),
        out_shape=jax.ShapeDtypeStruct((e_total, d), jnp.float32),
    )(h_g, ex.reshape(e_total, 1), s_g.reshape(e_total, 1))


def _pool_head_body(raw_ref, b2_ref, w1_ref, b1_ref, w2_ref, bo_ref, acc_ref, out_ref):
    i = pl.program_id(0)
    nsteps = pl.num_programs(0)
    blk_max = jnp.max(raw_ref[...], axis=0, keepdims=True)

    @pl.when(i == 0)
    def _():
        acc_ref[...] = blk_max

    @pl.when(i > 0)
    def _():
        acc_ref[...] = jnp.maximum(acc_ref[...], blk_max)

    @pl.when(i == nsteps - 1)
    def _():
        pooled = jnp.maximum(acc_ref[...] + b2_ref[...], 0.0)
        z = jnp.maximum(
            jnp.dot(pooled, w1_ref[...], preferred_element_type=jnp.float32, precision=jax.lax.Precision.HIGHEST) + b1_ref[...],
            0.0,
        )
        out_ref[...] = (
            jnp.dot(z, w2_ref[...], preferred_element_type=jnp.float32, precision=jax.lax.Precision.HIGHEST) + bo_ref[...]
        )


def _pool_head(raw3, b2, lin1_W, lin1_b, lin2_W, lin2_b):
    n, d = raw3.shape
    hid = lin1_W.shape[1]
    grid = n // N_BLK
    acc, out = pl.pallas_call(
        _pool_head_body,
        grid=(grid,),
        in_specs=[
            pl.BlockSpec((N_BLK, d), lambda i: (i, 0)),
            pl.BlockSpec((1, d), lambda i: (0, 0)),
            pl.BlockSpec((d, hid), lambda i: (0, 0)),
            pl.BlockSpec((1, hid), lambda i: (0, 0)),
            pl.BlockSpec((hid, 1), lambda i: (0, 0)),
            pl.BlockSpec((1, 1), lambda i: (0, 0)),
        ],
        out_specs=[
            pl.BlockSpec((1, d), lambda i: (0, 0)),
            pl.BlockSpec((1, 1), lambda i: (0, 0)),
        ],
        out_shape=[
            jax.ShapeDtypeStruct((1, d), jnp.float32),
            jax.ShapeDtypeStruct((1, 1), jnp.float32),
        ],
    )(raw3, b2.reshape(1, d), lin1_W, lin1_b.reshape(1, hid), lin2_W, lin2_b.reshape(1, 1))
    del acc
    return out


def _gat_layer(xin, src, dst, W, a_src, a_dst, b_prev, apply_relu, n):
    h, al, ar = _dense(xin, W, a_src, a_dst, b_prev, apply_relu)
    al = al.reshape(-1)
    ar = ar.reshape(-1)
    alg = jnp.take(al, src)
    arg = jnp.take(ar, dst)
    e = _edge_elementwise(_edge_logit_body, alg, arg)
    m = jax.ops.segment_max(e, dst, num_segments=n)
    m = jnp.where(jnp.isfinite(m), m, 0.0)
    ex = _edge_elementwise(_edge_exp_body, e, jnp.take(m, dst))
    s = jax.ops.segment_sum(ex, dst, num_segments=n)
    h_g = jnp.take(h, src, axis=0)
    msg = _messages(h_g, ex, jnp.take(s, dst))
    return jax.ops.segment_sum(msg, dst, num_segments=n)


def kernel(x, edge_index, batch, W1, a_src1, a_dst1, b1, W2, a_src2, a_dst2, b2,
           lin1_W, lin1_b, lin2_W, lin2_b):
    n = x.shape[0]
    loops = jnp.arange(n, dtype=edge_index.dtype)
    src = jnp.concatenate([edge_index[0], loops])
    dst = jnp.concatenate([edge_index[1], loops])

    zero_b = jnp.zeros_like(b1)
    raw1 = _gat_layer(x, src, dst, W1, a_src1, a_dst1, zero_b, False, n)
    raw2 = _gat_layer(raw1, src, dst, W2, a_src2, a_dst2, b1, True, n)
    raw3 = _gat_layer(raw2, src, dst, W2, a_src2, a_dst2, b2, True, n)
    return _pool_head(raw3, b2, lin1_W, lin1_b, lin2_W, lin2_b)
